# Initial kernel scaffold; baseline (speedup 1.0000x reference)
import jax
import jax.numpy as jnp
from jax.experimental import pallas as pl


def kernel(inputs):
    raise NotImplementedError("write your pallas kernel here")


# speedup vs baseline: 2.8841x; 2.8841x over previous
import math

import numpy as np
import jax
import jax.numpy as jnp
from jax.experimental import pallas as pl
from jax.experimental.pallas import tpu as pltpu

_D_MODEL = 768
_MAX_LEN = 1600


def _make_pe_np():
    pe = np.zeros((_MAX_LEN, _D_MODEL), dtype=np.float32)
    position = np.arange(0, _MAX_LEN, dtype=np.float32)[:, None]
    div_term = np.exp(
        np.arange(0, _D_MODEL, 2, dtype=np.float32) * (-math.log(10000.0) / _D_MODEL)
    )
    pe[:, 0::2] = np.sin(position * div_term)
    pe[:, 1::2] = np.cos(position * div_term)
    return pe


def _top3_kernel(first_ref, out_ref):
    f = first_ref[...]
    s = f.shape[0]
    soft = jnp.sum(jnp.abs(f[:, None, :] - f[None, :, :]), axis=-1)
    col = jax.lax.broadcasted_iota(jnp.int32, (s, s), 1)
    big = jnp.int32(2**30)
    for k in range(3):
        minv = jnp.min(soft, axis=1, keepdims=True)
        idx = jnp.min(jnp.where(soft == minv, col, big), axis=1)
        out_ref[:, k : k + 1] = idx[:, None]
        soft = jnp.where(col == idx[:, None], jnp.inf, soft)


def _attn_kernel(idx_ref, in0_ref, in1_ref, in2_ref, pe_ref, out_ref):
    x0 = in0_ref[0] + pe_ref[0]
    x1 = in1_ref[0] + pe_ref[1]
    x2 = in2_ref[0] + pe_ref[2]
    h = x0.shape[1]
    scale = 1.0 / math.sqrt(h)

    def mm_nt(a, b):
        return jax.lax.dot_general(
            a, b, (((1,), (1,)), ((), ())), preferred_element_type=jnp.float32
        )

    def mm_nn(a, b):
        return jax.lax.dot_general(
            a, b, (((1,), (0,)), ((), ())), preferred_element_type=jnp.float32
        )

    q = x0
    scores = jnp.concatenate([mm_nt(q, x0), mm_nt(q, x1), mm_nt(q, x2)], axis=1)
    scores = scores * scale
    m = jnp.max(scores, axis=1, keepdims=True)
    e = jnp.exp(scores - m)
    p = e / jnp.sum(e, axis=1, keepdims=True)
    w = x0.shape[0]
    out = mm_nn(p[:, :w], x0) + mm_nn(p[:, w : 2 * w], x1) + mm_nn(p[:, 2 * w :], x2)
    out_ref[0] = out


def kernel(inputs):
    sentence, word, hidden = inputs.shape
    first = inputs[:, 0, :]

    top3_padded = pl.pallas_call(
        _top3_kernel,
        out_shape=jax.ShapeDtypeStruct((sentence, 128), jnp.int32),
    )(first)
    top3_flat = top3_padded[:, :3].reshape(3 * sentence)

    pe3 = jnp.asarray(_make_pe_np()[: 3 * word].reshape(3, word, hidden))

    grid_spec = pltpu.PrefetchScalarGridSpec(
        num_scalar_prefetch=1,
        grid=(sentence,),
        in_specs=[
            pl.BlockSpec((1, word, hidden), lambda i, idx: (idx[3 * i], 0, 0)),
            pl.BlockSpec((1, word, hidden), lambda i, idx: (idx[3 * i + 1], 0, 0)),
            pl.BlockSpec((1, word, hidden), lambda i, idx: (idx[3 * i + 2], 0, 0)),
            pl.BlockSpec((3, word, hidden), lambda i, idx: (0, 0, 0)),
        ],
        out_specs=pl.BlockSpec((1, word, hidden), lambda i, idx: (i, 0, 0)),
    )
    fused = pl.pallas_call(
        _attn_kernel,
        grid_spec=grid_spec,
        out_shape=jax.ShapeDtypeStruct((sentence, word, hidden), jnp.float32),
    )(top3_flat, inputs, inputs, inputs, pe3)
    return fused[:, : word - 1, :]


# speedup vs baseline: 2.8863x; 1.0008x over previous
import math

import numpy as np
import jax
import jax.numpy as jnp
from jax.experimental import pallas as pl
from jax.experimental.pallas import tpu as pltpu

_D_MODEL = 768
_MAX_LEN = 1600


def _make_pe_np():
    pe = np.zeros((_MAX_LEN, _D_MODEL), dtype=np.float32)
    position = np.arange(0, _MAX_LEN, dtype=np.float32)[:, None]
    div_term = np.exp(
        np.arange(0, _D_MODEL, 2, dtype=np.float32) * (-math.log(10000.0) / _D_MODEL)
    )
    pe[:, 0::2] = np.sin(position * div_term)
    pe[:, 1::2] = np.cos(position * div_term)
    return pe


def _top3_kernel(first_ref, out_ref):
    f = first_ref[:, 0, :]
    s = f.shape[0]
    soft = jnp.sum(jnp.abs(f[:, None, :] - f[None, :, :]), axis=-1)
    col = jax.lax.broadcasted_iota(jnp.int32, (s, s), 1)
    big = jnp.int32(2**30)
    for k in range(3):
        minv = jnp.min(soft, axis=1, keepdims=True)
        idx = jnp.min(jnp.where(soft == minv, col, big), axis=1)
        out_ref[:, k : k + 1] = idx[:, None]
        soft = jnp.where(col == idx[:, None], jnp.inf, soft)


def _attn_kernel(idx_ref, in_ref, pe_ref, out_ref):
    i = pl.program_id(0)
    x0 = in_ref[idx_ref[i, 0]] + pe_ref[0]
    x1 = in_ref[idx_ref[i, 1]] + pe_ref[1]
    x2 = in_ref[idx_ref[i, 2]] + pe_ref[2]
    h = x0.shape[1]
    scale = 1.0 / math.sqrt(h)

    def mm_nt(a, b):
        return jax.lax.dot_general(
            a, b, (((1,), (1,)), ((), ())), preferred_element_type=jnp.float32
        )

    def mm_nn(a, b):
        return jax.lax.dot_general(
            a, b, (((1,), (0,)), ((), ())), preferred_element_type=jnp.float32
        )

    q = x0
    scores = jnp.concatenate([mm_nt(q, x0), mm_nt(q, x1), mm_nt(q, x2)], axis=1)
    scores = scores * scale
    m = jnp.max(scores, axis=1, keepdims=True)
    e = jnp.exp(scores - m)
    p = e / jnp.sum(e, axis=1, keepdims=True)
    w = x0.shape[0]
    out = mm_nn(p[:, :w], x0) + mm_nn(p[:, w : 2 * w], x1) + mm_nn(p[:, 2 * w :], x2)
    out_ref[0] = out


def kernel(inputs):
    sentence, word, hidden = inputs.shape

    top3 = pl.pallas_call(
        _top3_kernel,
        grid=(1,),
        in_specs=[pl.BlockSpec((sentence, 8, hidden), lambda i: (0, 0, 0))],
        out_specs=pl.BlockSpec((sentence, 128), lambda i: (0, 0)),
        out_shape=jax.ShapeDtypeStruct((sentence, 128), jnp.int32),
    )(inputs)

    pe3 = jnp.asarray(_make_pe_np()[: 3 * word].reshape(3, word, hidden))

    grid_spec = pltpu.PrefetchScalarGridSpec(
        num_scalar_prefetch=1,
        grid=(sentence,),
        in_specs=[
            pl.BlockSpec((sentence, word, hidden), lambda i, idx: (0, 0, 0)),
            pl.BlockSpec((3, word, hidden), lambda i, idx: (0, 0, 0)),
        ],
        out_specs=pl.BlockSpec((1, word, hidden), lambda i, idx: (i, 0, 0)),
    )
    fused = pl.pallas_call(
        _attn_kernel,
        grid_spec=grid_spec,
        out_shape=jax.ShapeDtypeStruct((sentence, word, hidden), jnp.float32),
    )(top3, inputs, pe3)
    return fused[:, : word - 1, :]


# speedup vs baseline: 3.1931x; 1.1063x over previous
import math

import numpy as np
import jax
import jax.numpy as jnp
from jax.experimental import pallas as pl
from jax.experimental.pallas import tpu as pltpu

_D_MODEL = 768
_MAX_LEN = 1600


def _make_pe_np():
    pe = np.zeros((_MAX_LEN, _D_MODEL), dtype=np.float32)
    position = np.arange(0, _MAX_LEN, dtype=np.float32)[:, None]
    div_term = np.exp(
        np.arange(0, _D_MODEL, 2, dtype=np.float32) * (-math.log(10000.0) / _D_MODEL)
    )
    pe[:, 0::2] = np.sin(position * div_term)
    pe[:, 1::2] = np.cos(position * div_term)
    return pe


def _top3_kernel(first_ref, out_ref):
    f = first_ref[:, 0, :]
    s = f.shape[0]
    soft = jnp.sum(jnp.abs(f[:, None, :] - f[None, :, :]), axis=-1)
    col = jax.lax.broadcasted_iota(jnp.int32, (s, s), 1)
    big = jnp.int32(2**30)
    for k in range(3):
        minv = jnp.min(soft, axis=1, keepdims=True)
        idx = jnp.min(jnp.where(soft == minv, col, big), axis=1)
        out_ref[:, k : k + 1] = idx[:, None]
        soft = jnp.where(col == idx[:, None], jnp.inf, soft)


def _attn_kernel(idx_ref, in_ref, pe_ref, out_ref):
    i = pl.program_id(0)
    x0 = in_ref[idx_ref[i, 0]] + pe_ref[0]
    x1 = in_ref[idx_ref[i, 1]] + pe_ref[1]
    x2 = in_ref[idx_ref[i, 2]] + pe_ref[2]
    h = x0.shape[1]
    scale = 1.0 / math.sqrt(h)

    def mm_nt(a, b):
        return jax.lax.dot_general(
            a, b, (((1,), (1,)), ((), ())), preferred_element_type=jnp.float32
        )

    def mm_nn(a, b):
        return jax.lax.dot_general(
            a, b, (((1,), (0,)), ((), ())), preferred_element_type=jnp.float32
        )

    q = x0
    scores = jnp.concatenate([mm_nt(q, x0), mm_nt(q, x1), mm_nt(q, x2)], axis=1)
    scores = scores * scale
    m = jnp.max(scores, axis=1, keepdims=True)
    e = jnp.exp(scores - m)
    p = e / jnp.sum(e, axis=1, keepdims=True)
    w = x0.shape[0]
    out = mm_nn(p[:, :w], x0) + mm_nn(p[:, w : 2 * w], x1) + mm_nn(p[:, 2 * w :], x2)
    out_ref[0] = out[: out_ref.shape[1]]


def kernel(inputs):
    sentence, word, hidden = inputs.shape

    top3 = pl.pallas_call(
        _top3_kernel,
        grid=(1,),
        in_specs=[pl.BlockSpec((sentence, 8, hidden), lambda i: (0, 0, 0))],
        out_specs=pl.BlockSpec((sentence, 128), lambda i: (0, 0)),
        out_shape=jax.ShapeDtypeStruct((sentence, 128), jnp.int32),
    )(inputs)

    pe3 = jnp.asarray(_make_pe_np()[: 3 * word].reshape(3, word, hidden))

    grid_spec = pltpu.PrefetchScalarGridSpec(
        num_scalar_prefetch=1,
        grid=(sentence,),
        in_specs=[
            pl.BlockSpec((sentence, word, hidden), lambda i, idx: (0, 0, 0)),
            pl.BlockSpec((3, word, hidden), lambda i, idx: (0, 0, 0)),
        ],
        out_specs=pl.BlockSpec((1, word - 1, hidden), lambda i, idx: (i, 0, 0)),
    )
    return pl.pallas_call(
        _attn_kernel,
        grid_spec=grid_spec,
        out_shape=jax.ShapeDtypeStruct((sentence, word - 1, hidden), jnp.float32),
    )(top3, inputs, pe3)


# speedup vs baseline: 3.2403x; 1.0148x over previous
import math

import numpy as np
import jax
import jax.numpy as jnp
from jax.experimental import pallas as pl
from jax.experimental.pallas import tpu as pltpu

_D_MODEL = 768
_MAX_LEN = 1600


def _make_pe_np():
    pe = np.zeros((_MAX_LEN, _D_MODEL), dtype=np.float32)
    position = np.arange(0, _MAX_LEN, dtype=np.float32)[:, None]
    div_term = np.exp(
        np.arange(0, _D_MODEL, 2, dtype=np.float32) * (-math.log(10000.0) / _D_MODEL)
    )
    pe[:, 0::2] = np.sin(position * div_term)
    pe[:, 1::2] = np.cos(position * div_term)
    return pe


def _top3_kernel(first_ref, out_ref):
    f = first_ref[:, 0, :]
    s = f.shape[0]
    soft = jnp.sum(jnp.abs(f[:, None, :] - f[None, :, :]), axis=-1)
    col = jax.lax.broadcasted_iota(jnp.int32, (s, s), 1)
    big = jnp.int32(2**30)
    for k in range(3):
        minv = jnp.min(soft, axis=1, keepdims=True)
        idx = jnp.min(jnp.where(soft == minv, col, big), axis=1)
        out_ref[:, k : k + 1] = idx[:, None]
        soft = jnp.where(col == idx[:, None], jnp.inf, soft)


def _attn_kernel(idx_ref, in_ref, pe_ref, out_ref):
    i = pl.program_id(0)
    x0 = in_ref[idx_ref[i, 0]] + pe_ref[0]
    x1 = in_ref[idx_ref[i, 1]] + pe_ref[1]
    x2 = in_ref[idx_ref[i, 2]] + pe_ref[2]
    h = x0.shape[1]
    scale = 1.0 / math.sqrt(h)

    def mm_nt(a, b):
        return jax.lax.dot_general(
            a, b, (((1,), (1,)), ((), ())), preferred_element_type=jnp.float32
        )

    def mm_nn(a, b):
        return jax.lax.dot_general(
            a, b, (((1,), (0,)), ((), ())), preferred_element_type=jnp.float32
        )

    x0b = x0.astype(jnp.bfloat16)
    x1b = x1.astype(jnp.bfloat16)
    x2b = x2.astype(jnp.bfloat16)
    q = x0b
    scores = jnp.concatenate([mm_nt(q, x0b), mm_nt(q, x1b), mm_nt(q, x2b)], axis=1)
    scores = scores * scale
    m = jnp.max(scores, axis=1, keepdims=True)
    e = jnp.exp(scores - m)
    rinv = 1.0 / jnp.sum(e, axis=1, keepdims=True)
    eb = e.astype(jnp.bfloat16)
    w = x0.shape[0]
    out = mm_nn(eb[:, :w], x0b) + mm_nn(eb[:, w : 2 * w], x1b) + mm_nn(eb[:, 2 * w :], x2b)
    out_ref[0] = (out * rinv)[: out_ref.shape[1]]


def kernel(inputs):
    sentence, word, hidden = inputs.shape

    top3 = pl.pallas_call(
        _top3_kernel,
        grid=(1,),
        in_specs=[pl.BlockSpec((sentence, 8, hidden), lambda i: (0, 0, 0))],
        out_specs=pl.BlockSpec((sentence, 128), lambda i: (0, 0)),
        out_shape=jax.ShapeDtypeStruct((sentence, 128), jnp.int32),
    )(inputs)

    pe3 = jnp.asarray(_make_pe_np()[: 3 * word].reshape(3, word, hidden))

    grid_spec = pltpu.PrefetchScalarGridSpec(
        num_scalar_prefetch=1,
        grid=(sentence,),
        in_specs=[
            pl.BlockSpec((sentence, word, hidden), lambda i, idx: (0, 0, 0)),
            pl.BlockSpec((3, word, hidden), lambda i, idx: (0, 0, 0)),
        ],
        out_specs=pl.BlockSpec((1, word - 1, hidden), lambda i, idx: (i, 0, 0)),
    )
    return pl.pallas_call(
        _attn_kernel,
        grid_spec=grid_spec,
        out_shape=jax.ShapeDtypeStruct((sentence, word - 1, hidden), jnp.float32),
    )(top3, inputs, pe3)


# speedup vs baseline: 3.6512x; 1.1268x over previous
import math

import numpy as np
import jax
import jax.numpy as jnp
from jax.experimental import pallas as pl
from jax.experimental.pallas import tpu as pltpu

_D_MODEL = 768
_MAX_LEN = 1600


def _make_pe_np():
    pe = np.zeros((_MAX_LEN, _D_MODEL), dtype=np.float32)
    position = np.arange(0, _MAX_LEN, dtype=np.float32)[:, None]
    div_term = np.exp(
        np.arange(0, _D_MODEL, 2, dtype=np.float32) * (-math.log(10000.0) / _D_MODEL)
    )
    pe[:, 0::2] = np.sin(position * div_term)
    pe[:, 1::2] = np.cos(position * div_term)
    return pe


def _top3_kernel(first_ref, out_ref):
    f = first_ref[:, 0, :]
    s = f.shape[0]
    soft = jnp.sum(jnp.abs(f[:, None, :] - f[None, :, :]), axis=-1)
    col = jax.lax.broadcasted_iota(jnp.int32, (s, s), 1)
    big = jnp.int32(2**30)
    for k in range(3):
        minv = jnp.min(soft, axis=1, keepdims=True)
        idx = jnp.min(jnp.where(soft == minv, col, big), axis=1)
        out_ref[:, k : k + 1] = idx[:, None]
        soft = jnp.where(col == idx[:, None], jnp.inf, soft)


def _attn_kernel(idx_ref, in_ref, pe_ref, out_ref, xb_ref):
    i = pl.program_id(0)
    w = in_ref.shape[1]
    h = in_ref.shape[2]
    scale = 1.0 / math.sqrt(h)
    for s in range(3):
        xs = in_ref[idx_ref[i, s]] + pe_ref[s]
        xb_ref[s * w : (s + 1) * w] = xs.astype(jnp.bfloat16)
    xb = xb_ref[...]
    q = xb[:w]
    scores = jax.lax.dot_general(
        q, xb, (((1,), (1,)), ((), ())), preferred_element_type=jnp.float32
    )
    scores = scores * scale
    m = jnp.max(scores, axis=1, keepdims=True)
    e = jnp.exp(scores - m)
    rinv = 1.0 / jnp.sum(e, axis=1, keepdims=True)
    eb = e.astype(jnp.bfloat16)
    out = jax.lax.dot_general(
        eb, xb, (((1,), (0,)), ((), ())), preferred_element_type=jnp.float32
    )
    out_ref[0] = (out * rinv)[: out_ref.shape[1]]


def kernel(inputs):
    sentence, word, hidden = inputs.shape

    top3 = pl.pallas_call(
        _top3_kernel,
        grid=(1,),
        in_specs=[pl.BlockSpec((sentence, 8, hidden), lambda i: (0, 0, 0))],
        out_specs=pl.BlockSpec((sentence, 128), lambda i: (0, 0)),
        out_shape=jax.ShapeDtypeStruct((sentence, 128), jnp.int32),
    )(inputs)

    pe3 = jnp.asarray(_make_pe_np()[: 3 * word].reshape(3, word, hidden))

    grid_spec = pltpu.PrefetchScalarGridSpec(
        num_scalar_prefetch=1,
        grid=(sentence,),
        in_specs=[
            pl.BlockSpec((sentence, word, hidden), lambda i, idx: (0, 0, 0)),
            pl.BlockSpec((3, word, hidden), lambda i, idx: (0, 0, 0)),
        ],
        out_specs=pl.BlockSpec((1, word - 1, hidden), lambda i, idx: (i, 0, 0)),
        scratch_shapes=[pltpu.VMEM((3 * word, hidden), jnp.bfloat16)],
    )
    return pl.pallas_call(
        _attn_kernel,
        grid_spec=grid_spec,
        out_shape=jax.ShapeDtypeStruct((sentence, word - 1, hidden), jnp.float32),
    )(top3, inputs, pe3)


# speedup vs baseline: 4.0223x; 1.1016x over previous
import math

import numpy as np
import jax
import jax.numpy as jnp
from jax.experimental import pallas as pl
from jax.experimental.pallas import tpu as pltpu

_D_MODEL = 768
_MAX_LEN = 1600


def _make_pe_np():
    pe = np.zeros((_MAX_LEN, _D_MODEL), dtype=np.float32)
    position = np.arange(0, _MAX_LEN, dtype=np.float32)[:, None]
    div_term = np.exp(
        np.arange(0, _D_MODEL, 2, dtype=np.float32) * (-math.log(10000.0) / _D_MODEL)
    )
    pe[:, 0::2] = np.sin(position * div_term)
    pe[:, 1::2] = np.cos(position * div_term)
    return pe


def _top3_kernel(first_ref, out_ref):
    f = first_ref[:, 0, :]
    s = f.shape[0]
    soft = jnp.sum(jnp.abs(f[:, None, :] - f[None, :, :]), axis=-1)
    col = jax.lax.broadcasted_iota(jnp.int32, (s, s), 1)
    big = jnp.int32(2**30)
    for k in range(3):
        minv = jnp.min(soft, axis=1, keepdims=True)
        idx = jnp.min(jnp.where(soft == minv, col, big), axis=1)
        out_ref[:, k : k + 1] = idx[:, None]
        soft = jnp.where(col == idx[:, None], jnp.inf, soft)


def _attn_kernel(idx_ref, in_ref, pe_ref, out_ref, xb_ref):
    i = pl.program_id(0)
    n_per = out_ref.shape[0]
    w = in_ref.shape[1]
    h = in_ref.shape[2]
    scale = 1.0 / math.sqrt(h)
    for j in range(n_per):
        for s in range(3):
            xs = in_ref[idx_ref[n_per * i + j, s]] + pe_ref[s]
            xb_ref[j, s * w : (s + 1) * w] = xs.astype(jnp.bfloat16)
    for j in range(n_per):
        xb = xb_ref[j]
        q = xb[:w]
        scores = jax.lax.dot_general(
            q, xb, (((1,), (1,)), ((), ())), preferred_element_type=jnp.float32
        )
        scores = scores * scale
        m = jnp.max(scores, axis=1, keepdims=True)
        e = jnp.exp(scores - m)
        rinv = 1.0 / jnp.sum(e, axis=1, keepdims=True)
        eb = e.astype(jnp.bfloat16)
        out = jax.lax.dot_general(
            eb, xb, (((1,), (0,)), ((), ())), preferred_element_type=jnp.float32
        )
        out_ref[j] = (out * rinv)[: out_ref.shape[1]]


def kernel(inputs):
    sentence, word, hidden = inputs.shape

    top3 = pl.pallas_call(
        _top3_kernel,
        grid=(1,),
        in_specs=[pl.BlockSpec((sentence, 8, hidden), lambda i: (0, 0, 0))],
        out_specs=pl.BlockSpec((sentence, 128), lambda i: (0, 0)),
        out_shape=jax.ShapeDtypeStruct((sentence, 128), jnp.int32),
    )(inputs)

    pe3 = jnp.asarray(_make_pe_np()[: 3 * word].reshape(3, word, hidden))

    n_per = 2
    grid_spec = pltpu.PrefetchScalarGridSpec(
        num_scalar_prefetch=1,
        grid=(sentence // n_per,),
        in_specs=[
            pl.BlockSpec((sentence, word, hidden), lambda i, idx: (0, 0, 0)),
            pl.BlockSpec((3, word, hidden), lambda i, idx: (0, 0, 0)),
        ],
        out_specs=pl.BlockSpec((n_per, word - 1, hidden), lambda i, idx: (i, 0, 0)),
        scratch_shapes=[pltpu.VMEM((n_per, 3 * word, hidden), jnp.bfloat16)],
    )
    return pl.pallas_call(
        _attn_kernel,
        grid_spec=grid_spec,
        out_shape=jax.ShapeDtypeStruct((sentence, word - 1, hidden), jnp.float32),
    )(top3, inputs, pe3)


# speedup vs baseline: 4.1996x; 1.0441x over previous
import math

import numpy as np
import jax
import jax.numpy as jnp
from jax.experimental import pallas as pl
from jax.experimental.pallas import tpu as pltpu

_D_MODEL = 768
_MAX_LEN = 1600


def _make_pe_np():
    pe = np.zeros((_MAX_LEN, _D_MODEL), dtype=np.float32)
    position = np.arange(0, _MAX_LEN, dtype=np.float32)[:, None]
    div_term = np.exp(
        np.arange(0, _D_MODEL, 2, dtype=np.float32) * (-math.log(10000.0) / _D_MODEL)
    )
    pe[:, 0::2] = np.sin(position * div_term)
    pe[:, 1::2] = np.cos(position * div_term)
    return pe


def _top3_kernel(first_ref, out_ref):
    f = first_ref[:, 0, :]
    s = f.shape[0]
    soft = jnp.sum(jnp.abs(f[:, None, :] - f[None, :, :]), axis=-1)
    col = jax.lax.broadcasted_iota(jnp.int32, (s, s), 1)
    big = jnp.int32(2**30)
    for k in range(3):
        minv = jnp.min(soft, axis=1, keepdims=True)
        idx = jnp.min(jnp.where(soft == minv, col, big), axis=1)
        out_ref[:, k : k + 1] = idx[:, None]
        soft = jnp.where(col == idx[:, None], jnp.inf, soft)


def _attn_kernel(idx_ref, in_ref, pe_ref, out_ref, xb_ref):
    i = pl.program_id(0)
    n_per = out_ref.shape[0]
    w = in_ref.shape[1]
    h = in_ref.shape[2]
    scale = 1.0 / math.sqrt(h)
    for j in range(n_per):
        for s in range(3):
            xs = in_ref[idx_ref[n_per * i + j, s]] + pe_ref[s]
            xb_ref[j, s * w : (s + 1) * w] = xs.astype(jnp.bfloat16)
    for j in range(n_per):
        xb = xb_ref[j]
        q = xb[:w]
        scores = jax.lax.dot_general(
            q, xb, (((1,), (1,)), ((), ())), preferred_element_type=jnp.float32
        )
        scores = scores * scale
        m = jnp.max(scores, axis=1, keepdims=True)
        e = jnp.exp(scores - m)
        rinv = 1.0 / jnp.sum(e, axis=1, keepdims=True)
        eb = e.astype(jnp.bfloat16)
        out = jax.lax.dot_general(
            eb, xb, (((1,), (0,)), ((), ())), preferred_element_type=jnp.float32
        )
        out_ref[j] = (out * rinv)[: out_ref.shape[1]]


def kernel(inputs):
    sentence, word, hidden = inputs.shape

    top3 = pl.pallas_call(
        _top3_kernel,
        grid=(1,),
        in_specs=[pl.BlockSpec((sentence, 8, hidden), lambda i: (0, 0, 0))],
        out_specs=pl.BlockSpec((sentence, 128), lambda i: (0, 0)),
        out_shape=jax.ShapeDtypeStruct((sentence, 128), jnp.int32),
    )(inputs)

    pe3 = jnp.asarray(_make_pe_np()[: 3 * word].reshape(3, word, hidden))

    n_per = 4
    grid_spec = pltpu.PrefetchScalarGridSpec(
        num_scalar_prefetch=1,
        grid=(sentence // n_per,),
        in_specs=[
            pl.BlockSpec((sentence, word, hidden), lambda i, idx: (0, 0, 0)),
            pl.BlockSpec((3, word, hidden), lambda i, idx: (0, 0, 0)),
        ],
        out_specs=pl.BlockSpec((n_per, word - 1, hidden), lambda i, idx: (i, 0, 0)),
        scratch_shapes=[pltpu.VMEM((n_per, 3 * word, hidden), jnp.bfloat16)],
    )
    return pl.pallas_call(
        _attn_kernel,
        grid_spec=grid_spec,
        out_shape=jax.ShapeDtypeStruct((sentence, word - 1, hidden), jnp.float32),
    )(top3, inputs, pe3)


# speedup vs baseline: 4.2488x; 1.0117x over previous
import math

import numpy as np
import jax
import jax.numpy as jnp
from jax.experimental import pallas as pl
from jax.experimental.pallas import tpu as pltpu

_D_MODEL = 768
_MAX_LEN = 1600


def _make_pe_np():
    pe = np.zeros((_MAX_LEN, _D_MODEL), dtype=np.float32)
    position = np.arange(0, _MAX_LEN, dtype=np.float32)[:, None]
    div_term = np.exp(
        np.arange(0, _D_MODEL, 2, dtype=np.float32) * (-math.log(10000.0) / _D_MODEL)
    )
    pe[:, 0::2] = np.sin(position * div_term)
    pe[:, 1::2] = np.cos(position * div_term)
    return pe


def _top3_kernel(first_ref, out_ref):
    f = first_ref[:, 0, :]
    s = f.shape[0]
    soft = jnp.sum(jnp.abs(f[:, None, :] - f[None, :, :]), axis=-1)
    col = jax.lax.broadcasted_iota(jnp.int32, (s, s), 1)
    big = jnp.int32(2**30)
    for k in range(3):
        minv = jnp.min(soft, axis=1, keepdims=True)
        idx = jnp.min(jnp.where(soft == minv, col, big), axis=1)
        out_ref[:, k : k + 1] = idx[:, None]
        soft = jnp.where(col == idx[:, None], jnp.inf, soft)


def _attn_kernel(idx_ref, in_ref, pe_ref, out_ref, xb_ref):
    i = pl.program_id(0)
    n_per = out_ref.shape[0]
    w = in_ref.shape[1]
    h = in_ref.shape[2]
    scale = 1.0 / math.sqrt(h)
    for j in range(n_per):
        for s in range(3):
            xs = in_ref[idx_ref[n_per * i + j, s]] + pe_ref[s]
            xb_ref[j, s * w : (s + 1) * w] = xs.astype(jnp.bfloat16)
    for j in range(n_per):
        xb = xb_ref[j]
        q = xb[:w]
        scores = jax.lax.dot_general(
            q, xb, (((1,), (1,)), ((), ())), preferred_element_type=jnp.float32
        )
        scores = scores * scale
        m = jnp.max(scores, axis=1, keepdims=True)
        e = jnp.exp(scores - m)
        rinv = 1.0 / jnp.sum(e, axis=1, keepdims=True)
        eb = e.astype(jnp.bfloat16)
        out = jax.lax.dot_general(
            eb, xb, (((1,), (0,)), ((), ())), preferred_element_type=jnp.float32
        )
        out_ref[j] = (out * rinv)[: out_ref.shape[1]]


def kernel(inputs):
    sentence, word, hidden = inputs.shape

    top3 = pl.pallas_call(
        _top3_kernel,
        grid=(1,),
        in_specs=[pl.BlockSpec((sentence, 8, hidden), lambda i: (0, 0, 0))],
        out_specs=pl.BlockSpec((sentence, 128), lambda i: (0, 0)),
        out_shape=jax.ShapeDtypeStruct((sentence, 128), jnp.int32),
    )(inputs)

    pe3 = jnp.asarray(_make_pe_np()[: 3 * word].reshape(3, word, hidden))

    n_per = 8
    grid_spec = pltpu.PrefetchScalarGridSpec(
        num_scalar_prefetch=1,
        grid=(sentence // n_per,),
        in_specs=[
            pl.BlockSpec((sentence, word, hidden), lambda i, idx: (0, 0, 0)),
            pl.BlockSpec((3, word, hidden), lambda i, idx: (0, 0, 0)),
        ],
        out_specs=pl.BlockSpec((n_per, word - 1, hidden), lambda i, idx: (i, 0, 0)),
        scratch_shapes=[pltpu.VMEM((n_per, 3 * word, hidden), jnp.bfloat16)],
    )
    return pl.pallas_call(
        _attn_kernel,
        grid_spec=grid_spec,
        out_shape=jax.ShapeDtypeStruct((sentence, word - 1, hidden), jnp.float32),
    )(top3, inputs, pe3)
